# Initial kernel scaffold; baseline (speedup 1.0000x reference)
import jax
import jax.numpy as jnp
from jax.experimental import pallas as pl


def kernel(sequences, expert_weights, expert_biases, gate_w, gate_b):
    raise NotImplementedError("write your pallas kernel here")


# speedup vs baseline: 23.5255x; 23.5255x over previous
import functools

import jax
import jax.numpy as jnp
from jax.experimental import pallas as pl
from jax.experimental.pallas import tpu as pltpu

N_EXP = 64
D_MODEL = 768
D_INNER = 256


def _moe_dense_body(x_ref, gw_ref, gb_ref, w_ref, b_ref, out_ref,
                    i0_ref, i1_ref, m0_ref, m1_ref):
    e = pl.program_id(0)

    @pl.when(e == 0)
    def _gating():
        x = x_ref[...]
        logits = jnp.dot(x, gw_ref[...], preferred_element_type=jnp.float32)
        logits = logits + gb_ref[...]
        mx = jnp.max(logits, axis=1, keepdims=True)
        ex = jnp.exp(logits - mx)
        probs = ex / jnp.sum(ex, axis=1, keepdims=True)
        iota = jax.lax.broadcasted_iota(jnp.int32, probs.shape, 1)
        m0 = jnp.max(probs, axis=1, keepdims=True)
        i0 = jnp.min(jnp.where(probs == m0, iota, N_EXP), axis=1, keepdims=True)
        masked = jnp.where(iota == i0, -jnp.inf, probs)
        m1 = jnp.max(masked, axis=1, keepdims=True)
        i1 = jnp.min(jnp.where(masked == m1, iota, N_EXP), axis=1, keepdims=True)
        i0_ref[...] = i0
        i1_ref[...] = i1
        m0_ref[...] = m0 * 0.5
        m1_ref[...] = m1 * 0.5
        out_ref[...] = jnp.zeros_like(out_ref)

    coef = (jnp.where(i0_ref[...] == e, m0_ref[...], 0.0)
            + jnp.where(i1_ref[...] == e, m1_ref[...], 0.0))
    y = jnp.dot(x_ref[...], w_ref[0], preferred_element_type=jnp.float32)
    y = y + b_ref[0]
    out_ref[...] += coef * y


def kernel(sequences, expert_weights, expert_biases, gate_w, gate_b):
    n, s, d = sequences.shape
    x = sequences.reshape(n * s, d)
    t = n * s
    gb2 = gate_b.reshape(1, N_EXP)

    out = pl.pallas_call(
        _moe_dense_body,
        grid=(N_EXP,),
        in_specs=[
            pl.BlockSpec((t, D_MODEL), lambda e: (0, 0)),
            pl.BlockSpec((D_MODEL, N_EXP), lambda e: (0, 0)),
            pl.BlockSpec((1, N_EXP), lambda e: (0, 0)),
            pl.BlockSpec((1, D_MODEL, D_INNER), lambda e: (e, 0, 0)),
            pl.BlockSpec((1, 1, D_INNER), lambda e: (e, 0, 0)),
        ],
        out_specs=pl.BlockSpec((t, D_INNER), lambda e: (0, 0)),
        out_shape=jax.ShapeDtypeStruct((t, D_INNER), jnp.float32),
        scratch_shapes=[
            pltpu.VMEM((t, 1), jnp.int32),
            pltpu.VMEM((t, 1), jnp.int32),
            pltpu.VMEM((t, 1), jnp.float32),
            pltpu.VMEM((t, 1), jnp.float32),
        ],
        compiler_params=pltpu.CompilerParams(
            dimension_semantics=("arbitrary",),
        ),
    )(x, gate_w, gb2, expert_weights, expert_biases.reshape(N_EXP, 1, D_INNER))
    return out.reshape(n, s, D_INNER)
